# Initial kernel scaffold; baseline (speedup 1.0000x reference)
import jax
import jax.numpy as jnp
from jax.experimental import pallas as pl


def kernel(x, adj, W1, b1, W2, b2):
    raise NotImplementedError("write your pallas kernel here")


# speedup vs baseline: 7.5244x; 7.5244x over previous
import functools

import jax
import jax.numpy as jnp
from jax import lax
from jax.experimental import pallas as pl
from jax.experimental.pallas import tpu as pltpu
from jax.experimental.pallas import tpu_sc as plsc

N = 10000
E = 320000
NFEAT = 128
NHID = 256
NCLASS = 64

NC = 2
NS = 16
LANES = 128

NP = 10112
RPS = NP // NS
EP = 327680
NBLK = EP // LANES

_mesh = plsc.VectorSubcoreMesh(core_axis_name="c", subcore_axis_name="s")


_HBLK = NBLK // (NC * NS)


@functools.partial(
    pl.kernel,
    out_type=jax.ShapeDtypeStruct((2 * NP, LANES), jnp.float32),
    mesh=_mesh,
    scratch_types=[
        pltpu.VMEM((_HBLK, LANES), jnp.int32),
        pltpu.VMEM((LANES, LANES), jnp.float32),
        pltpu.VMEM_SHARED((NP, LANES), jnp.float32),
        pltpu.SemaphoreType.DMA,
    ],
)
def _sc_degree(dst_hbm, out_hbm, idxv, onesv, acc, sem):
    c = lax.axis_index("c")
    s = lax.axis_index("s")
    wid = c * NS + s

    @pl.loop(0, LANES)
    def _(i):
        @pl.loop(0, LANES, step=16)
        def _(k):
            onesv[i, pl.ds(k, 16)] = jnp.ones((16,), jnp.float32)

    @pl.loop(0, RPS, step=8)
    def _(r):
        pltpu.sync_copy(onesv.at[pl.ds(0, 8)], acc.at[pl.ds(s * RPS + r, 8)])

    pltpu.sync_copy(dst_hbm.at[pl.ds(wid * _HBLK, _HBLK)], idxv)
    plsc.subcore_barrier()

    @pl.loop(0, _HBLK)
    def _(i):
        pltpu.sync_copy(onesv, acc.at[idxv.at[i]], add=True)

    plsc.subcore_barrier()
    pltpu.sync_copy(
        acc.at[pl.ds(s * RPS, RPS)], out_hbm.at[pl.ds(c * NP + s * RPS, RPS)]
    )


_CH = 16


def _make_agg(H, edge_split):
    if edge_split:
        wblk = NBLK // (NC * NS)
    else:
        wblk = NBLK // NS
    nch = wblk // _CH

    @functools.partial(
        pl.kernel,
        out_type=jax.ShapeDtypeStruct((2 * NP, H), jnp.float32),
        mesh=_mesh,
        scratch_types=[
            pltpu.VMEM((_CH, LANES), jnp.int32),
            pltpu.VMEM((_CH, LANES), jnp.int32),
            pltpu.VMEM((LANES, H), jnp.float32),
            pltpu.VMEM((LANES, H), jnp.float32),
            pltpu.VMEM_SHARED((NP, H), jnp.float32),
            pltpu.SemaphoreType.DMA,
            pltpu.SemaphoreType.DMA,
        ],
    )
    def agg(g_hbm, src_hbm, dst_hbm, out_hbm,
            sv, dv, rows0, rows1, acc, sem0, sem1):
        c = lax.axis_index("c")
        s = lax.axis_index("s")

        pltpu.sync_copy(g_hbm.at[pl.ds(c * NP + s * RPS, RPS)],
                        acc.at[pl.ds(s * RPS, RPS)])
        plsc.subcore_barrier()

        @pl.loop(0, nch)
        def _(ci):
            if edge_split:
                blk0 = (c * NS + s) * wblk + ci * _CH
            else:
                blk0 = s * wblk + ci * _CH
            src0 = c * NBLK + blk0
            pltpu.sync_copy(src_hbm.at[pl.ds(src0, _CH)], sv)
            pltpu.sync_copy(dst_hbm.at[pl.ds(blk0, _CH)], dv)

            @pl.loop(0, _CH // 2)
            def _(j):
                i = j * 2
                cp0 = pltpu.async_copy(g_hbm.at[sv.at[i]], rows0, sem0)
                cp1 = pltpu.async_copy(g_hbm.at[sv.at[i + 1]], rows1, sem1)
                cp0.wait()
                pltpu.sync_copy(rows0, acc.at[dv.at[i]], add=True)
                cp1.wait()
                pltpu.sync_copy(rows1, acc.at[dv.at[i + 1]], add=True)

        plsc.subcore_barrier()
        pltpu.sync_copy(acc.at[pl.ds(s * RPS, RPS)],
                        out_hbm.at[pl.ds(c * NP + s * RPS, RPS)])

    return agg


_agg1 = _make_agg(NHID // 2, edge_split=False)
_agg2 = _make_agg(128, edge_split=True)


def _tc1_body(h0_ref, h1_ref, x_ref, w1_ref, ga_ref, gb_ref, dinv_ref):
    deg = h0_ref[:, 0:1] + h1_ref[:, 0:1] - 1.0
    dinv = lax.rsqrt(deg)
    h = jnp.dot(x_ref[...], w1_ref[...], precision=lax.Precision.HIGHEST,
                preferred_element_type=jnp.float32)
    g = h * dinv
    ga_ref[...] = g[:, : NHID // 2]
    gb_ref[...] = g[:, NHID // 2:]
    dinv_ref[...] = dinv


def _tc2_body(aa_ref, ab_ref, dinv_ref, b1_ref, w2_ref, g2_ref):
    agg = jnp.concatenate([aa_ref[...], ab_ref[...]], axis=1)
    dinv = dinv_ref[...]
    h = jnp.maximum(agg * dinv + b1_ref[...], 0.0)
    h2 = jnp.dot(h, w2_ref[...], precision=lax.Precision.HIGHEST,
                 preferred_element_type=jnp.float32)
    g2 = h2 * dinv
    g2_ref[...] = jnp.concatenate([g2, jnp.zeros_like(g2)], axis=1)


def _tc3_body(p0_ref, p1_ref, g2_ref, dinv_ref, b2_ref, out_ref):
    z = (p0_ref[:, :NCLASS] + p1_ref[:, :NCLASS] - g2_ref[:, :NCLASS])
    z = z * dinv_ref[...] + b2_ref[...]
    m = jnp.max(z, axis=1, keepdims=True)
    e = jnp.exp(z - m)
    lse = jnp.log(jnp.sum(e, axis=1, keepdims=True)) + m
    out_ref[...] = z - lse


_f32 = jnp.float32
_MG = 8
_MB = NP // _MG


def _row_spec(w):
    return pl.BlockSpec((_MB, w), lambda i: (i, 0))


def _full_spec(r, w):
    return pl.BlockSpec((r, w), lambda i: (0, 0))


_tc1 = pl.pallas_call(
    _tc1_body,
    grid=(_MG,),
    in_specs=[_row_spec(LANES), _row_spec(LANES), _row_spec(NFEAT),
              _full_spec(NFEAT, NHID)],
    out_specs=(_row_spec(NHID // 2), _row_spec(NHID // 2), _row_spec(1)),
    out_shape=(
        jax.ShapeDtypeStruct((NP, NHID // 2), _f32),
        jax.ShapeDtypeStruct((NP, NHID // 2), _f32),
        jax.ShapeDtypeStruct((NP, 1), _f32),
    ),
)

_tc2 = pl.pallas_call(
    _tc2_body,
    grid=(_MG,),
    in_specs=[_row_spec(NHID // 2), _row_spec(NHID // 2), _row_spec(1),
              _full_spec(1, NHID), _full_spec(NHID, NCLASS)],
    out_specs=_row_spec(128),
    out_shape=jax.ShapeDtypeStruct((NP, 128), _f32),
)

_tc3 = pl.pallas_call(
    _tc3_body,
    grid=(_MG,),
    in_specs=[_row_spec(128), _row_spec(128), _row_spec(128), _row_spec(1),
              _full_spec(1, NCLASS)],
    out_specs=_row_spec(NCLASS),
    out_shape=jax.ShapeDtypeStruct((NP, NCLASS), _f32),
)


@jax.jit
def kernel(x, adj, W1, b1, W2, b2):
    src = adj[0]
    dst = adj[1]
    pad = jnp.full((EP - E,), N, dtype=jnp.int32)
    src2 = jnp.concatenate([src, pad]).reshape(NBLK, LANES)
    dst2 = jnp.concatenate([dst, pad]).reshape(NBLK, LANES)
    src2s = jnp.concatenate([src2, src2 + NP], axis=0)

    xp = jnp.concatenate([x, jnp.zeros((NP - N, NFEAT), _f32)], axis=0)

    hist = _sc_degree(dst2)
    g1a, g1b, dinv = _tc1(hist[:NP], hist[NP:], xp, W1)
    a1 = _agg1(jnp.concatenate([g1a, g1b], axis=0), src2s, dst2)
    g2 = _tc2(a1[:NP], a1[NP:], dinv, b1.reshape(1, NHID), W2)
    a2 = _agg2(jnp.concatenate([g2, g2], axis=0), src2s, dst2)
    out = _tc3(a2[:NP], a2[NP:], g2, dinv, b2.reshape(1, NCLASS))
    return out[:N]


# speedup vs baseline: 17.1778x; 2.2829x over previous
import functools

import jax
import jax.numpy as jnp
from jax import lax
from jax.experimental import pallas as pl
from jax.experimental.pallas import tpu as pltpu
from jax.experimental.pallas import tpu_sc as plsc

N = 10000
E = 320000
NFEAT = 128
NHID = 256
NCLASS = 64

NC = 2
NS = 16
LANES = 128

NP = 10112
RPS = NP // NS
EP = 327680
NBLK = EP // LANES

_mesh = plsc.VectorSubcoreMesh(core_axis_name="c", subcore_axis_name="s")


_HBLK = NBLK // (NC * NS)


@functools.partial(
    pl.kernel,
    out_type=jax.ShapeDtypeStruct((2 * NP, LANES), jnp.float32),
    mesh=_mesh,
    scratch_types=[
        pltpu.VMEM((_HBLK, LANES), jnp.int32),
        pltpu.VMEM((LANES, LANES), jnp.float32),
        pltpu.VMEM_SHARED((NP, LANES), jnp.float32),
        pltpu.SemaphoreType.DMA,
    ],
)
def _sc_degree(dst_hbm, out_hbm, idxv, onesv, acc, sem):
    c = lax.axis_index("c")
    s = lax.axis_index("s")
    wid = c * NS + s

    @pl.loop(0, LANES)
    def _(i):
        @pl.loop(0, LANES, step=16)
        def _(k):
            onesv[i, pl.ds(k, 16)] = jnp.ones((16,), jnp.float32)

    @pl.loop(0, RPS, step=8)
    def _(r):
        pltpu.sync_copy(onesv.at[pl.ds(0, 8)], acc.at[pl.ds(s * RPS + r, 8)])

    pltpu.sync_copy(dst_hbm.at[pl.ds(wid * _HBLK, _HBLK)], idxv)
    plsc.subcore_barrier()

    @pl.loop(0, _HBLK)
    def _(i):
        pltpu.sync_copy(onesv, acc.at[idxv.at[i]], add=True)

    plsc.subcore_barrier()
    pltpu.sync_copy(
        acc.at[pl.ds(s * RPS, RPS)], out_hbm.at[pl.ds(c * NP + s * RPS, RPS)]
    )


_CH = 16


def _make_agg(H, edge_split):
    if edge_split:
        wblk = NBLK // (NC * NS)
    else:
        wblk = NBLK // NS
    nch = wblk // _CH

    @functools.partial(
        pl.kernel,
        out_type=jax.ShapeDtypeStruct((2 * NP, H), jnp.float32),
        mesh=_mesh,
        scratch_types=[
            pltpu.VMEM((_CH, LANES), jnp.int32),
            pltpu.VMEM((_CH, LANES), jnp.int32),
            pltpu.VMEM((LANES, H), jnp.float32),
            pltpu.VMEM((LANES, H), jnp.float32),
            pltpu.VMEM_SHARED((NP, H), jnp.float32),
            pltpu.SemaphoreType.DMA,
            pltpu.SemaphoreType.DMA,
        ],
    )
    def agg(g_hbm, src_hbm, dst_hbm, out_hbm,
            sv, dv, rows0, rows1, acc, sem0, sem1):
        c = lax.axis_index("c")
        s = lax.axis_index("s")

        pltpu.sync_copy(g_hbm.at[pl.ds(c * NP + s * RPS, RPS)],
                        acc.at[pl.ds(s * RPS, RPS)])
        plsc.subcore_barrier()

        @pl.loop(0, nch)
        def _(ci):
            if edge_split:
                blk0 = (c * NS + s) * wblk + ci * _CH
            else:
                blk0 = s * wblk + ci * _CH
            src0 = c * NBLK + blk0
            pltpu.sync_copy(src_hbm.at[pl.ds(src0, _CH)], sv)
            pltpu.sync_copy(dst_hbm.at[pl.ds(blk0, _CH)], dv)

            @pl.loop(0, _CH // 2)
            def _(j):
                i = j * 2
                cp0 = pltpu.async_copy(g_hbm.at[sv.at[i]], rows0, sem0)
                cp1 = pltpu.async_copy(g_hbm.at[sv.at[i + 1]], rows1, sem1)
                cp0.wait()
                pltpu.sync_copy(rows0, acc.at[dv.at[i]], add=True)
                cp1.wait()
                pltpu.sync_copy(rows1, acc.at[dv.at[i + 1]], add=True)

        plsc.subcore_barrier()
        pltpu.sync_copy(acc.at[pl.ds(s * RPS, RPS)],
                        out_hbm.at[pl.ds(c * NP + s * RPS, RPS)])

    return agg


_agg1 = _make_agg(NHID // 2, edge_split=False)
_agg2 = _make_agg(128, edge_split=True)


def _tc1_body(h0_ref, h1_ref, x_ref, w1_ref, ga_ref, gb_ref, dinv_ref):
    deg = h0_ref[:, 0:1] + h1_ref[:, 0:1] - 1.0
    dinv = lax.rsqrt(deg)
    h = jnp.dot(x_ref[...], w1_ref[...], precision=lax.Precision.HIGHEST,
                preferred_element_type=jnp.float32)
    g = h * dinv
    ga_ref[...] = g[:, : NHID // 2]
    gb_ref[...] = g[:, NHID // 2:]
    dinv_ref[...] = dinv


def _tc2_body(aa_ref, ab_ref, dinv_ref, b1_ref, w2_ref, g2_ref):
    agg = jnp.concatenate([aa_ref[...], ab_ref[...]], axis=1)
    dinv = dinv_ref[...]
    h = jnp.maximum(agg * dinv + b1_ref[...], 0.0)
    h2 = jnp.dot(h, w2_ref[...], precision=lax.Precision.HIGHEST,
                 preferred_element_type=jnp.float32)
    g2 = h2 * dinv
    g2_ref[...] = jnp.concatenate([g2, jnp.zeros_like(g2)], axis=1)


def _tc3_body(p0_ref, p1_ref, g2_ref, dinv_ref, b2_ref, out_ref):
    z = (p0_ref[:, :NCLASS] + p1_ref[:, :NCLASS] - g2_ref[:, :NCLASS])
    z = z * dinv_ref[...] + b2_ref[...]
    m = jnp.max(z, axis=1, keepdims=True)
    e = jnp.exp(z - m)
    lse = jnp.log(jnp.sum(e, axis=1, keepdims=True)) + m
    out_ref[...] = z - lse


_f32 = jnp.float32
_MG = 8
_MB = NP // _MG


def _row_spec(w):
    return pl.BlockSpec((_MB, w), lambda i: (i, 0))


def _full_spec(r, w):
    return pl.BlockSpec((r, w), lambda i: (0, 0))


_tc1 = pl.pallas_call(
    _tc1_body,
    grid=(_MG,),
    in_specs=[_row_spec(LANES), _row_spec(LANES), _row_spec(NFEAT),
              _full_spec(NFEAT, NHID)],
    out_specs=(_row_spec(NHID // 2), _row_spec(NHID // 2), _row_spec(1)),
    out_shape=(
        jax.ShapeDtypeStruct((NP, NHID // 2), _f32),
        jax.ShapeDtypeStruct((NP, NHID // 2), _f32),
        jax.ShapeDtypeStruct((NP, 1), _f32),
    ),
)

_tc2 = pl.pallas_call(
    _tc2_body,
    grid=(_MG,),
    in_specs=[_row_spec(NHID // 2), _row_spec(NHID // 2), _row_spec(1),
              _full_spec(1, NHID), _full_spec(NHID, NCLASS)],
    out_specs=_row_spec(128),
    out_shape=jax.ShapeDtypeStruct((NP, 128), _f32),
)

_tc3 = pl.pallas_call(
    _tc3_body,
    grid=(_MG,),
    in_specs=[_row_spec(128), _row_spec(128), _row_spec(128), _row_spec(1),
              _full_spec(1, NCLASS)],
    out_specs=_row_spec(NCLASS),
    out_shape=jax.ShapeDtypeStruct((NP, NCLASS), _f32),
)


@jax.jit
def kernel(x, adj, W1, b1, W2, b2):
    src = adj[0]
    dst = adj[1]
    pad = N + jnp.arange(EP - E, dtype=jnp.int32) % (NP - N)
    src2 = jnp.concatenate([src, pad]).reshape(NBLK, LANES)
    dst2 = jnp.concatenate([dst, pad]).reshape(NBLK, LANES)
    src2s = jnp.concatenate([src2, src2 + NP], axis=0)

    xp = jnp.concatenate([x, jnp.zeros((NP - N, NFEAT), _f32)], axis=0)

    hist = _sc_degree(dst2)
    g1a, g1b, dinv = _tc1(hist[:NP], hist[NP:], xp, W1)
    a1 = _agg1(jnp.concatenate([g1a, g1b], axis=0), src2s, dst2)
    g2 = _tc2(a1[:NP], a1[NP:], dinv, b1.reshape(1, NHID), W2)
    a2 = _agg2(jnp.concatenate([g2, g2], axis=0), src2s, dst2)
    out = _tc3(a2[:NP], a2[NP:], g2, dinv, b2.reshape(1, NCLASS))
    return out[:N]


# speedup vs baseline: 18.4302x; 1.0729x over previous
import functools

import jax
import jax.numpy as jnp
from jax import lax
from jax.experimental import pallas as pl
from jax.experimental.pallas import tpu as pltpu
from jax.experimental.pallas import tpu_sc as plsc

N = 10000
E = 320000
NFEAT = 128
NHID = 256
NCLASS = 64

NC = 2
NS = 16
LANES = 128

NP = 10112
RPS = NP // NS
EP = 327680
NBLK = EP // LANES

_mesh = plsc.VectorSubcoreMesh(core_axis_name="c", subcore_axis_name="s")


_HBLK = NBLK // (NC * NS)


HW = 16


@functools.partial(
    pl.kernel,
    out_type=jax.ShapeDtypeStruct((2 * NP, HW), jnp.float32),
    mesh=_mesh,
    scratch_types=[
        pltpu.VMEM((_HBLK, LANES), jnp.int32),
        pltpu.VMEM((LANES, HW), jnp.float32),
        pltpu.VMEM_SHARED((NP, HW), jnp.float32),
        pltpu.SemaphoreType.DMA,
    ],
)
def _sc_degree(dst_hbm, out_hbm, idxv, onesv, acc, sem):
    c = lax.axis_index("c")
    s = lax.axis_index("s")
    wid = c * NS + s

    @pl.loop(0, LANES)
    def _(i):
        onesv[i, pl.ds(0, HW)] = jnp.ones((HW,), jnp.float32)

    @pl.loop(0, RPS, step=8)
    def _(r):
        pltpu.sync_copy(onesv.at[pl.ds(0, 8)], acc.at[pl.ds(s * RPS + r, 8)])

    pltpu.sync_copy(dst_hbm.at[pl.ds(wid * _HBLK, _HBLK)], idxv)
    plsc.subcore_barrier()

    @pl.loop(0, _HBLK)
    def _(i):
        pltpu.sync_copy(onesv, acc.at[idxv.at[i]], add=True)

    plsc.subcore_barrier()
    pltpu.sync_copy(
        acc.at[pl.ds(s * RPS, RPS)], out_hbm.at[pl.ds(c * NP + s * RPS, RPS)]
    )


_CH = 16


def _make_agg(H, edge_split):
    if edge_split:
        wblk = NBLK // (NC * NS)
    else:
        wblk = NBLK // NS
    nch = wblk // _CH

    @functools.partial(
        pl.kernel,
        out_type=jax.ShapeDtypeStruct((2 * NP, H), jnp.float32),
        mesh=_mesh,
        scratch_types=[
            pltpu.VMEM((_CH, LANES), jnp.int32),
            pltpu.VMEM((_CH, LANES), jnp.int32),
            pltpu.VMEM((LANES, H), jnp.float32),
            pltpu.VMEM((LANES, H), jnp.float32),
            pltpu.VMEM_SHARED((NP, H), jnp.float32),
            pltpu.SemaphoreType.DMA,
            pltpu.SemaphoreType.DMA,
        ],
    )
    def agg(g_hbm, src_hbm, dst_hbm, out_hbm,
            sv, dv, rows0, rows1, acc, sem0, sem1):
        c = lax.axis_index("c")
        s = lax.axis_index("s")

        pltpu.sync_copy(g_hbm.at[pl.ds(c * NP + s * RPS, RPS)],
                        acc.at[pl.ds(s * RPS, RPS)])
        plsc.subcore_barrier()

        @pl.loop(0, nch)
        def _(ci):
            if edge_split:
                blk0 = (c * NS + s) * wblk + ci * _CH
            else:
                blk0 = s * wblk + ci * _CH
            src0 = c * NBLK + blk0
            pltpu.sync_copy(src_hbm.at[pl.ds(src0, _CH)], sv)
            pltpu.sync_copy(dst_hbm.at[pl.ds(blk0, _CH)], dv)

            @pl.loop(0, _CH // 2)
            def _(j):
                i = j * 2
                cp0 = pltpu.async_copy(g_hbm.at[sv.at[i]], rows0, sem0)
                cp1 = pltpu.async_copy(g_hbm.at[sv.at[i + 1]], rows1, sem1)
                cp0.wait()
                pltpu.sync_copy(rows0, acc.at[dv.at[i]], add=True)
                cp1.wait()
                pltpu.sync_copy(rows1, acc.at[dv.at[i + 1]], add=True)

        plsc.subcore_barrier()
        pltpu.sync_copy(acc.at[pl.ds(s * RPS, RPS)],
                        out_hbm.at[pl.ds(c * NP + s * RPS, RPS)])

    return agg


_agg1 = _make_agg(NHID // 2, edge_split=False)
_agg2 = _make_agg(128, edge_split=True)


def _tc1_body(h0_ref, h1_ref, x_ref, w1_ref, ga_ref, gb_ref, dinv_ref):
    deg = h0_ref[:, 0:1] + h1_ref[:, 0:1] - 1.0
    dinv = lax.rsqrt(deg)
    h = jnp.dot(x_ref[...], w1_ref[...], precision=lax.Precision.HIGHEST,
                preferred_element_type=jnp.float32)
    g = h * dinv
    ga_ref[...] = g[:, : NHID // 2]
    gb_ref[...] = g[:, NHID // 2:]
    dinv_ref[...] = dinv


def _tc2_body(aa_ref, ab_ref, dinv_ref, b1_ref, w2_ref, g2_ref):
    agg = jnp.concatenate([aa_ref[...], ab_ref[...]], axis=1)
    dinv = dinv_ref[...]
    h = jnp.maximum(agg * dinv + b1_ref[...], 0.0)
    h2 = jnp.dot(h, w2_ref[...], precision=lax.Precision.HIGHEST,
                 preferred_element_type=jnp.float32)
    g2 = h2 * dinv
    g2_ref[...] = jnp.concatenate([g2, jnp.zeros_like(g2)], axis=1)


def _tc3_body(p0_ref, p1_ref, g2_ref, dinv_ref, b2_ref, out_ref):
    z = (p0_ref[:, :NCLASS] + p1_ref[:, :NCLASS] - g2_ref[:, :NCLASS])
    z = z * dinv_ref[...] + b2_ref[...]
    m = jnp.max(z, axis=1, keepdims=True)
    e = jnp.exp(z - m)
    lse = jnp.log(jnp.sum(e, axis=1, keepdims=True)) + m
    out_ref[...] = z - lse


_f32 = jnp.float32
_MG = 8
_MB = NP // _MG


def _row_spec(w):
    return pl.BlockSpec((_MB, w), lambda i: (i, 0))


def _full_spec(r, w):
    return pl.BlockSpec((r, w), lambda i: (0, 0))


_tc1 = pl.pallas_call(
    _tc1_body,
    grid=(_MG,),
    in_specs=[_row_spec(HW), _row_spec(HW), _row_spec(NFEAT),
              _full_spec(NFEAT, NHID)],
    out_specs=(_row_spec(NHID // 2), _row_spec(NHID // 2), _row_spec(1)),
    out_shape=(
        jax.ShapeDtypeStruct((NP, NHID // 2), _f32),
        jax.ShapeDtypeStruct((NP, NHID // 2), _f32),
        jax.ShapeDtypeStruct((NP, 1), _f32),
    ),
)

_tc2 = pl.pallas_call(
    _tc2_body,
    grid=(_MG,),
    in_specs=[_row_spec(NHID // 2), _row_spec(NHID // 2), _row_spec(1),
              _full_spec(1, NHID), _full_spec(NHID, NCLASS)],
    out_specs=_row_spec(128),
    out_shape=jax.ShapeDtypeStruct((NP, 128), _f32),
)

_tc3 = pl.pallas_call(
    _tc3_body,
    grid=(_MG,),
    in_specs=[_row_spec(128), _row_spec(128), _row_spec(128), _row_spec(1),
              _full_spec(1, NCLASS)],
    out_specs=_row_spec(NCLASS),
    out_shape=jax.ShapeDtypeStruct((NP, NCLASS), _f32),
)


@jax.jit
def kernel(x, adj, W1, b1, W2, b2):
    src = adj[0]
    dst = adj[1]
    pad = N + jnp.arange(EP - E, dtype=jnp.int32) % (NP - N)
    src2 = jnp.concatenate([src, pad]).reshape(NBLK, LANES)
    dst2 = jnp.concatenate([dst, pad]).reshape(NBLK, LANES)
    src2s = jnp.concatenate([src2, src2 + NP], axis=0)

    xp = jnp.concatenate([x, jnp.zeros((NP - N, NFEAT), _f32)], axis=0)

    hist = _sc_degree(dst2)
    g1a, g1b, dinv = _tc1(hist[:NP], hist[NP:], xp, W1)
    a1 = _agg1(jnp.concatenate([g1a, g1b], axis=0), src2s, dst2)
    g2 = _tc2(a1[:NP], a1[NP:], dinv, b1.reshape(1, NHID), W2)
    a2 = _agg2(jnp.concatenate([g2, g2], axis=0), src2s, dst2)
    out = _tc3(a2[:NP], a2[NP:], g2, dinv, b2.reshape(1, NCLASS))
    return out[:N]


# speedup vs baseline: 19.0721x; 1.0348x over previous
import functools

import jax
import jax.numpy as jnp
from jax import lax
from jax.experimental import pallas as pl
from jax.experimental.pallas import tpu as pltpu
from jax.experimental.pallas import tpu_sc as plsc

N = 10000
E = 320000
NFEAT = 128
NHID = 256
NCLASS = 64

NC = 2
NS = 16
LANES = 128

NP = 10112
RPS = NP // NS
EP = 327680
NBLK = EP // LANES

_mesh = plsc.VectorSubcoreMesh(core_axis_name="c", subcore_axis_name="s")


_HBLK = NBLK // (NC * NS)


HW = 16


@functools.partial(
    pl.kernel,
    out_type=jax.ShapeDtypeStruct((2 * NP, HW), jnp.float32),
    mesh=_mesh,
    scratch_types=[
        pltpu.VMEM((_HBLK, LANES), jnp.int32),
        pltpu.VMEM((LANES, HW), jnp.float32),
        pltpu.VMEM_SHARED((NP, HW), jnp.float32),
        pltpu.SemaphoreType.DMA,
    ],
)
def _sc_degree(dst_hbm, out_hbm, idxv, onesv, acc, sem):
    c = lax.axis_index("c")
    s = lax.axis_index("s")
    wid = c * NS + s

    @pl.loop(0, LANES)
    def _(i):
        onesv[i, pl.ds(0, HW)] = jnp.ones((HW,), jnp.float32)

    @pl.loop(0, RPS, step=8)
    def _(r):
        pltpu.sync_copy(onesv.at[pl.ds(0, 8)], acc.at[pl.ds(s * RPS + r, 8)])

    pltpu.sync_copy(dst_hbm.at[pl.ds(wid * _HBLK, _HBLK)], idxv)
    plsc.subcore_barrier()

    @pl.loop(0, _HBLK)
    def _(i):
        pltpu.sync_copy(onesv, acc.at[idxv.at[i]], add=True)

    plsc.subcore_barrier()
    pltpu.sync_copy(
        acc.at[pl.ds(s * RPS, RPS)], out_hbm.at[pl.ds(c * NP + s * RPS, RPS)]
    )


BL = 64
NBLK2 = EP // BL
_CH = 32
_NB = 4


def _make_agg(H, edge_split):
    if edge_split:
        wblk = NBLK2 // (NC * NS)
    else:
        wblk = NBLK2 // NS
    nch = wblk // _CH
    ngrp = _CH // _NB

    @functools.partial(
        pl.kernel,
        out_type=jax.ShapeDtypeStruct((2 * NP, H), jnp.float32),
        mesh=_mesh,
        scratch_types=[
            pltpu.VMEM((_CH, BL), jnp.int32),
            pltpu.VMEM((_CH, BL), jnp.int32),
            pltpu.VMEM((BL, H), jnp.float32),
            pltpu.VMEM((BL, H), jnp.float32),
            pltpu.VMEM((BL, H), jnp.float32),
            pltpu.VMEM((BL, H), jnp.float32),
            pltpu.VMEM_SHARED((NP, H), jnp.float32),
            pltpu.SemaphoreType.DMA,
            pltpu.SemaphoreType.DMA,
            pltpu.SemaphoreType.DMA,
            pltpu.SemaphoreType.DMA,
            pltpu.SemaphoreType.DMA,
            pltpu.SemaphoreType.DMA,
            pltpu.SemaphoreType.DMA,
            pltpu.SemaphoreType.DMA,
        ],
    )
    def agg(g_hbm, src_hbm, dst_hbm, out_hbm,
            sv, dv, rows0, rows1, rows2, rows3, acc,
            sg0, sg1, sg2, sg3, ss0, ss1, ss2, ss3):
        c = lax.axis_index("c")
        s = lax.axis_index("s")
        rows = (rows0, rows1, rows2, rows3)
        sg = (sg0, sg1, sg2, sg3)
        ss = (ss0, ss1, ss2, ss3)

        pltpu.sync_copy(g_hbm.at[pl.ds(c * NP + s * RPS, RPS)],
                        acc.at[pl.ds(s * RPS, RPS)])
        plsc.subcore_barrier()

        @pl.loop(0, nch)
        def _(ci):
            if edge_split:
                blk0 = (c * NS + s) * wblk + ci * _CH
            else:
                blk0 = s * wblk + ci * _CH
            src0 = c * NBLK2 + blk0
            pltpu.sync_copy(src_hbm.at[pl.ds(src0, _CH)], sv)
            pltpu.sync_copy(dst_hbm.at[pl.ds(blk0, _CH)], dv)

            @pl.loop(0, ngrp)
            def _(gi):
                @pl.when((ci > 0) | (gi > 0))
                def _():
                    for b in range(_NB):
                        pltpu.make_async_copy(
                            g_hbm.at[pl.ds(0, BL)], rows[b], ss[b]).wait()

                cps = []
                for b in range(_NB):
                    cps.append(pltpu.async_copy(
                        g_hbm.at[sv.at[gi * _NB + b]], rows[b], sg[b]))
                for b in range(_NB):
                    cps[b].wait()
                    pltpu.async_copy(
                        rows[b], acc.at[dv.at[gi * _NB + b]], ss[b], add=True)

        for b in range(_NB):
            pltpu.make_async_copy(g_hbm.at[pl.ds(0, BL)], rows[b], ss[b]).wait()
        plsc.subcore_barrier()
        pltpu.sync_copy(acc.at[pl.ds(s * RPS, RPS)],
                        out_hbm.at[pl.ds(c * NP + s * RPS, RPS)])

    return agg


_agg1 = _make_agg(NHID // 2, edge_split=False)
_agg2 = _make_agg(128, edge_split=True)


def _tc1_body(h0_ref, h1_ref, x_ref, w1_ref, ga_ref, gb_ref, dinv_ref):
    deg = h0_ref[:, 0:1] + h1_ref[:, 0:1] - 1.0
    dinv = lax.rsqrt(deg)
    h = jnp.dot(x_ref[...], w1_ref[...], precision=lax.Precision.HIGHEST,
                preferred_element_type=jnp.float32)
    g = h * dinv
    ga_ref[...] = g[:, : NHID // 2]
    gb_ref[...] = g[:, NHID // 2:]
    dinv_ref[...] = dinv


def _tc2_body(aa_ref, ab_ref, dinv_ref, b1_ref, w2_ref, g2_ref):
    agg = jnp.concatenate([aa_ref[...], ab_ref[...]], axis=1)
    dinv = dinv_ref[...]
    h = jnp.maximum(agg * dinv + b1_ref[...], 0.0)
    h2 = jnp.dot(h, w2_ref[...], precision=lax.Precision.HIGHEST,
                 preferred_element_type=jnp.float32)
    g2 = h2 * dinv
    g2_ref[...] = jnp.concatenate([g2, jnp.zeros_like(g2)], axis=1)


def _tc3_body(p0_ref, p1_ref, g2_ref, dinv_ref, b2_ref, out_ref):
    z = (p0_ref[:, :NCLASS] + p1_ref[:, :NCLASS] - g2_ref[:, :NCLASS])
    z = z * dinv_ref[...] + b2_ref[...]
    m = jnp.max(z, axis=1, keepdims=True)
    e = jnp.exp(z - m)
    lse = jnp.log(jnp.sum(e, axis=1, keepdims=True)) + m
    out_ref[...] = z - lse


_f32 = jnp.float32
_MG = 8
_MB = NP // _MG


def _row_spec(w):
    return pl.BlockSpec((_MB, w), lambda i: (i, 0))


def _full_spec(r, w):
    return pl.BlockSpec((r, w), lambda i: (0, 0))


_tc1 = pl.pallas_call(
    _tc1_body,
    grid=(_MG,),
    in_specs=[_row_spec(HW), _row_spec(HW), _row_spec(NFEAT),
              _full_spec(NFEAT, NHID)],
    out_specs=(_row_spec(NHID // 2), _row_spec(NHID // 2), _row_spec(1)),
    out_shape=(
        jax.ShapeDtypeStruct((NP, NHID // 2), _f32),
        jax.ShapeDtypeStruct((NP, NHID // 2), _f32),
        jax.ShapeDtypeStruct((NP, 1), _f32),
    ),
)

_tc2 = pl.pallas_call(
    _tc2_body,
    grid=(_MG,),
    in_specs=[_row_spec(NHID // 2), _row_spec(NHID // 2), _row_spec(1),
              _full_spec(1, NHID), _full_spec(NHID, NCLASS)],
    out_specs=_row_spec(128),
    out_shape=jax.ShapeDtypeStruct((NP, 128), _f32),
)

_tc3 = pl.pallas_call(
    _tc3_body,
    grid=(_MG,),
    in_specs=[_row_spec(128), _row_spec(128), _row_spec(128), _row_spec(1),
              _full_spec(1, NCLASS)],
    out_specs=_row_spec(NCLASS),
    out_shape=jax.ShapeDtypeStruct((NP, NCLASS), _f32),
)


@jax.jit
def kernel(x, adj, W1, b1, W2, b2):
    src = adj[0]
    dst = adj[1]
    pad = N + jnp.arange(EP - E, dtype=jnp.int32) % (NP - N)
    src2 = jnp.concatenate([src, pad]).reshape(NBLK, LANES)
    dst2 = jnp.concatenate([dst, pad]).reshape(NBLK, LANES)
    src2s = jnp.concatenate([src2, src2 + NP], axis=0)
    dst2b = dst2.reshape(NBLK2, BL)
    src2sb = src2s.reshape(2 * NBLK2, BL)

    xp = jnp.concatenate([x, jnp.zeros((NP - N, NFEAT), _f32)], axis=0)

    hist = _sc_degree(dst2)
    g1a, g1b, dinv = _tc1(hist[:NP], hist[NP:], xp, W1)
    a1 = _agg1(jnp.concatenate([g1a, g1b], axis=0), src2sb, dst2b)
    g2 = _tc2(a1[:NP], a1[NP:], dinv, b1.reshape(1, NHID), W2)
    a2 = _agg2(jnp.concatenate([g2, g2], axis=0), src2sb, dst2b)
    out = _tc3(a2[:NP], a2[NP:], g2, dinv, b2.reshape(1, NCLASS))
    return out[:N]


# speedup vs baseline: 23.9225x; 1.2543x over previous
import functools

import jax
import jax.numpy as jnp
from jax import lax
from jax.experimental import pallas as pl
from jax.experimental.pallas import tpu as pltpu
from jax.experimental.pallas import tpu_sc as plsc

N = 10000
E = 320000
NFEAT = 128
NHID = 256
NCLASS = 64

NC = 2
NS = 16
LANES = 128

NP = 10112
RPS = NP // NS
EP = 327680
NBLK = EP // LANES

_mesh = plsc.VectorSubcoreMesh(core_axis_name="c", subcore_axis_name="s")


_HBLK = NBLK // (NC * NS)


HW = 16


@functools.partial(
    pl.kernel,
    out_type=jax.ShapeDtypeStruct((2 * NP, HW), jnp.float32),
    mesh=_mesh,
    scratch_types=[
        pltpu.VMEM((_HBLK, LANES), jnp.int32),
        pltpu.VMEM((LANES, HW), jnp.float32),
        pltpu.VMEM_SHARED((NP, HW), jnp.float32),
        pltpu.SemaphoreType.DMA,
    ],
)
def _sc_degree(dst_hbm, out_hbm, idxv, onesv, acc, sem):
    c = lax.axis_index("c")
    s = lax.axis_index("s")
    wid = c * NS + s

    @pl.loop(0, LANES)
    def _(i):
        onesv[i, pl.ds(0, HW)] = jnp.ones((HW,), jnp.float32)

    @pl.loop(0, RPS, step=8)
    def _(r):
        pltpu.sync_copy(onesv.at[pl.ds(0, 8)], acc.at[pl.ds(s * RPS + r, 8)])

    pltpu.sync_copy(dst_hbm.at[pl.ds(wid * _HBLK, _HBLK)], idxv)
    plsc.subcore_barrier()

    @pl.loop(0, _HBLK)
    def _(i):
        pltpu.sync_copy(onesv, acc.at[idxv.at[i]], add=True)

    plsc.subcore_barrier()
    pltpu.sync_copy(
        acc.at[pl.ds(s * RPS, RPS)], out_hbm.at[pl.ds(c * NP + s * RPS, RPS)]
    )


BL = 64
NBLK2 = EP // BL
_CH = 32
_NB = 4


def _make_agg(H, edge_split):
    if edge_split:
        wblk = NBLK2 // (NC * NS)
    else:
        wblk = NBLK2 // NS
    nch = wblk // _CH
    ngrp = _CH // _NB

    @functools.partial(
        pl.kernel,
        out_type=jax.ShapeDtypeStruct((2 * NP, H), jnp.float32),
        mesh=_mesh,
        scratch_types=[
            pltpu.VMEM((_CH, BL), jnp.int32),
            pltpu.VMEM((_CH, BL), jnp.int32),
            pltpu.VMEM((BL, H), jnp.float32),
            pltpu.VMEM((BL, H), jnp.float32),
            pltpu.VMEM((BL, H), jnp.float32),
            pltpu.VMEM((BL, H), jnp.float32),
            pltpu.VMEM_SHARED((NP, H), jnp.float32),
            pltpu.SemaphoreType.DMA,
            pltpu.SemaphoreType.DMA,
            pltpu.SemaphoreType.DMA,
            pltpu.SemaphoreType.DMA,
            pltpu.SemaphoreType.DMA,
            pltpu.SemaphoreType.DMA,
            pltpu.SemaphoreType.DMA,
            pltpu.SemaphoreType.DMA,
        ],
    )
    def agg(g_hbm, src_hbm, dst_hbm, out_hbm,
            sv, dv, rows0, rows1, rows2, rows3, acc,
            sg0, sg1, sg2, sg3, ss0, ss1, ss2, ss3):
        c = lax.axis_index("c")
        s = lax.axis_index("s")
        rows = (rows0, rows1, rows2, rows3)
        sg = (sg0, sg1, sg2, sg3)
        ss = (ss0, ss1, ss2, ss3)

        pltpu.sync_copy(g_hbm.at[pl.ds(c * NP + s * RPS, RPS)],
                        acc.at[pl.ds(s * RPS, RPS)])
        plsc.subcore_barrier()

        @pl.loop(0, nch)
        def _(ci):
            if edge_split:
                blk0 = (c * NS + s) * wblk + ci * _CH
            else:
                blk0 = s * wblk + ci * _CH
            src0 = c * NBLK2 + blk0
            pltpu.sync_copy(src_hbm.at[pl.ds(src0, _CH)], sv)
            pltpu.sync_copy(dst_hbm.at[pl.ds(blk0, _CH)], dv)

            @pl.loop(0, ngrp)
            def _(gi):
                @pl.when((ci > 0) | (gi > 0))
                def _():
                    for b in range(_NB):
                        pltpu.make_async_copy(
                            g_hbm.at[pl.ds(0, BL)], rows[b], ss[b]).wait()

                cps = []
                for b in range(_NB):
                    cps.append(pltpu.async_copy(
                        g_hbm.at[sv.at[gi * _NB + b]], rows[b], sg[b]))
                for b in range(_NB):
                    cps[b].wait()
                    pltpu.async_copy(
                        rows[b], acc.at[dv.at[gi * _NB + b]], ss[b], add=True)

        for b in range(_NB):
            pltpu.make_async_copy(g_hbm.at[pl.ds(0, BL)], rows[b], ss[b]).wait()
        plsc.subcore_barrier()
        pltpu.sync_copy(acc.at[pl.ds(s * RPS, RPS)],
                        out_hbm.at[pl.ds(c * NP + s * RPS, RPS)])

    return agg


_agg = _make_agg(128, edge_split=True)


def _tc1_body(h0_ref, h1_ref, x_ref, gx_ref, dinv_ref):
    deg = h0_ref[:, 0:1] + h1_ref[:, 0:1] - 1.0
    dinv = lax.rsqrt(deg)
    gx_ref[...] = x_ref[...] * dinv
    dinv_ref[...] = dinv


def _tc2_body(p0_ref, p1_ref, gx_ref, dinv_ref, b1_ref, w1_ref, w2_ref,
              g2_ref):
    dinv = dinv_ref[...]
    agg = (p0_ref[...] + p1_ref[...] - gx_ref[...]) * dinv
    h1 = jnp.dot(agg, w1_ref[...], precision=lax.Precision.HIGHEST,
                 preferred_element_type=jnp.float32)
    h = jnp.maximum(h1 + b1_ref[...], 0.0)
    h2 = jnp.dot(h, w2_ref[...], precision=lax.Precision.HIGHEST,
                 preferred_element_type=jnp.float32)
    g2 = h2 * dinv
    g2_ref[...] = jnp.concatenate([g2, jnp.zeros_like(g2)], axis=1)


def _tc3_body(p0_ref, p1_ref, g2_ref, dinv_ref, b2_ref, out_ref):
    z = (p0_ref[:, :NCLASS] + p1_ref[:, :NCLASS] - g2_ref[:, :NCLASS])
    z = z * dinv_ref[...] + b2_ref[...]
    m = jnp.max(z, axis=1, keepdims=True)
    e = jnp.exp(z - m)
    lse = jnp.log(jnp.sum(e, axis=1, keepdims=True)) + m
    out_ref[...] = z - lse


_f32 = jnp.float32
_MG = 8
_MB = NP // _MG


def _row_spec(w):
    return pl.BlockSpec((_MB, w), lambda i: (i, 0))


def _full_spec(r, w):
    return pl.BlockSpec((r, w), lambda i: (0, 0))


_tc1 = pl.pallas_call(
    _tc1_body,
    grid=(_MG,),
    in_specs=[_row_spec(HW), _row_spec(HW), _row_spec(NFEAT)],
    out_specs=(_row_spec(NFEAT), _row_spec(1)),
    out_shape=(
        jax.ShapeDtypeStruct((NP, NFEAT), _f32),
        jax.ShapeDtypeStruct((NP, 1), _f32),
    ),
)

_tc2 = pl.pallas_call(
    _tc2_body,
    grid=(_MG,),
    in_specs=[_row_spec(NFEAT), _row_spec(NFEAT), _row_spec(NFEAT),
              _row_spec(1), _full_spec(1, NHID), _full_spec(NFEAT, NHID),
              _full_spec(NHID, NCLASS)],
    out_specs=_row_spec(128),
    out_shape=jax.ShapeDtypeStruct((NP, 128), _f32),
)

_tc3 = pl.pallas_call(
    _tc3_body,
    grid=(_MG,),
    in_specs=[_row_spec(128), _row_spec(128), _row_spec(128), _row_spec(1),
              _full_spec(1, NCLASS)],
    out_specs=_row_spec(NCLASS),
    out_shape=jax.ShapeDtypeStruct((NP, NCLASS), _f32),
)


@jax.jit
def kernel(x, adj, W1, b1, W2, b2):
    src = adj[0]
    dst = adj[1]
    pad = N + jnp.arange(EP - E, dtype=jnp.int32) % (NP - N)
    src2 = jnp.concatenate([src, pad]).reshape(NBLK, LANES)
    dst2 = jnp.concatenate([dst, pad]).reshape(NBLK, LANES)
    src2s = jnp.concatenate([src2, src2 + NP], axis=0)
    dst2b = dst2.reshape(NBLK2, BL)
    src2sb = src2s.reshape(2 * NBLK2, BL)

    xp = jnp.concatenate([x, jnp.zeros((NP - N, NFEAT), _f32)], axis=0)

    hist = _sc_degree(dst2)
    gx, dinv = _tc1(hist[:NP], hist[NP:], xp)
    a1 = _agg(jnp.concatenate([gx, gx], axis=0), src2sb, dst2b)
    g2 = _tc2(a1[:NP], a1[NP:], gx, dinv, b1.reshape(1, NHID), W1, W2)
    a2 = _agg(jnp.concatenate([g2, g2], axis=0), src2sb, dst2b)
    out = _tc3(a2[:NP], a2[NP:], g2, dinv, b2.reshape(1, NCLASS))
    return out[:N]


# speedup vs baseline: 24.9033x; 1.0410x over previous
import functools

import jax
import jax.numpy as jnp
from jax import lax
from jax.experimental import pallas as pl
from jax.experimental.pallas import tpu as pltpu
from jax.experimental.pallas import tpu_sc as plsc

N = 10000
E = 320000
NFEAT = 128
NHID = 256
NCLASS = 64

NC = 2
NS = 16
LANES = 128

NP = 10112
RPS = NP // NS
EP = 327680
NBLK = EP // LANES

_mesh = plsc.VectorSubcoreMesh(core_axis_name="c", subcore_axis_name="s")


_HBLK = NBLK // (NC * NS)


HW = 16


@functools.partial(
    pl.kernel,
    out_type=jax.ShapeDtypeStruct((2 * NP, HW), jnp.float32),
    mesh=_mesh,
    scratch_types=[
        pltpu.VMEM((_HBLK, LANES), jnp.int32),
        pltpu.VMEM((LANES, HW), jnp.float32),
        pltpu.VMEM_SHARED((NP, HW), jnp.float32),
        pltpu.SemaphoreType.DMA,
    ],
)
def _sc_degree(dst_hbm, out_hbm, idxv, onesv, acc, sem):
    c = lax.axis_index("c")
    s = lax.axis_index("s")
    wid = c * NS + s

    @pl.loop(0, LANES)
    def _(i):
        onesv[i, pl.ds(0, HW)] = jnp.ones((HW,), jnp.float32)

    @pl.loop(0, RPS, step=8)
    def _(r):
        pltpu.sync_copy(onesv.at[pl.ds(0, 8)], acc.at[pl.ds(s * RPS + r, 8)])

    pltpu.sync_copy(dst_hbm.at[pl.ds(wid * _HBLK, _HBLK)], idxv)
    plsc.subcore_barrier()

    @pl.loop(0, _HBLK)
    def _(i):
        pltpu.sync_copy(onesv, acc.at[idxv.at[i]], add=True)

    plsc.subcore_barrier()
    pltpu.sync_copy(
        acc.at[pl.ds(s * RPS, RPS)], out_hbm.at[pl.ds(c * NP + s * RPS, RPS)]
    )


BL = 64
NBLK2 = EP // BL
_CH = 32
_NB = 4


def _make_agg(H, edge_split):
    if edge_split:
        wblk = NBLK2 // (NC * NS)
    else:
        wblk = NBLK2 // NS
    nch = wblk // _CH
    ngrp = _CH // _NB

    @functools.partial(
        pl.kernel,
        out_type=jax.ShapeDtypeStruct((2 * NP, H), jnp.float32),
        mesh=_mesh,
        scratch_types=[
            pltpu.VMEM((_CH, BL), jnp.int32),
            pltpu.VMEM((_CH, BL), jnp.int32),
            pltpu.VMEM((BL, H), jnp.float32),
            pltpu.VMEM((BL, H), jnp.float32),
            pltpu.VMEM((BL, H), jnp.float32),
            pltpu.VMEM((BL, H), jnp.float32),
            pltpu.VMEM_SHARED((NP, H), jnp.float32),
            pltpu.SemaphoreType.DMA,
            pltpu.SemaphoreType.DMA,
            pltpu.SemaphoreType.DMA,
            pltpu.SemaphoreType.DMA,
            pltpu.SemaphoreType.DMA,
            pltpu.SemaphoreType.DMA,
            pltpu.SemaphoreType.DMA,
            pltpu.SemaphoreType.DMA,
        ],
    )
    def agg(g_hbm, src_hbm, dst_hbm, out_hbm,
            sv, dv, rows0, rows1, rows2, rows3, acc,
            sg0, sg1, sg2, sg3, ss0, ss1, ss2, ss3):
        c = lax.axis_index("c")
        s = lax.axis_index("s")
        rows = (rows0, rows1, rows2, rows3)
        sg = (sg0, sg1, sg2, sg3)
        ss = (ss0, ss1, ss2, ss3)

        pltpu.sync_copy(g_hbm.at[pl.ds(s * RPS, RPS)],
                        acc.at[pl.ds(s * RPS, RPS)])
        plsc.subcore_barrier()

        @pl.loop(0, nch)
        def _(ci):
            if edge_split:
                blk0 = (c * NS + s) * wblk + ci * _CH
            else:
                blk0 = s * wblk + ci * _CH
            pltpu.sync_copy(src_hbm.at[pl.ds(blk0, _CH)], sv)
            pltpu.sync_copy(dst_hbm.at[pl.ds(blk0, _CH)], dv)

            @pl.loop(0, ngrp)
            def _(gi):
                @pl.when((ci > 0) | (gi > 0))
                def _():
                    for b in range(_NB):
                        pltpu.make_async_copy(
                            g_hbm.at[pl.ds(0, BL)], rows[b], ss[b]).wait()

                cps = []
                for b in range(_NB):
                    cps.append(pltpu.async_copy(
                        g_hbm.at[sv.at[gi * _NB + b]], rows[b], sg[b]))
                for b in range(_NB):
                    cps[b].wait()
                    pltpu.async_copy(
                        rows[b], acc.at[dv.at[gi * _NB + b]], ss[b], add=True)

        for b in range(_NB):
            pltpu.make_async_copy(g_hbm.at[pl.ds(0, BL)], rows[b], ss[b]).wait()
        plsc.subcore_barrier()
        pltpu.sync_copy(acc.at[pl.ds(s * RPS, RPS)],
                        out_hbm.at[pl.ds(c * NP + s * RPS, RPS)])

    return agg


_agg = _make_agg(128, edge_split=True)


def _tc1_body(h0_ref, h1_ref, x_ref, gx_ref, dinv_ref):
    deg = h0_ref[:, 0:1] + h1_ref[:, 0:1] - 1.0
    dinv = lax.rsqrt(deg)
    gx_ref[...] = x_ref[...] * dinv
    dinv_ref[...] = dinv


def _tc2_body(p0_ref, p1_ref, gx_ref, dinv_ref, b1_ref, w1_ref, w2_ref,
              g2_ref):
    dinv = dinv_ref[...]
    agg = (p0_ref[...] + p1_ref[...] - gx_ref[...]) * dinv
    h1 = jnp.dot(agg, w1_ref[...], precision=lax.Precision.HIGHEST,
                 preferred_element_type=jnp.float32)
    h = jnp.maximum(h1 + b1_ref[...], 0.0)
    h2 = jnp.dot(h, w2_ref[...], precision=lax.Precision.HIGHEST,
                 preferred_element_type=jnp.float32)
    g2 = h2 * dinv
    g2_ref[...] = jnp.concatenate([g2, jnp.zeros_like(g2)], axis=1)


def _tc3_body(p0_ref, p1_ref, g2_ref, dinv_ref, b2_ref, out_ref):
    z = (p0_ref[:, :NCLASS] + p1_ref[:, :NCLASS] - g2_ref[:, :NCLASS])
    z = z * dinv_ref[...] + b2_ref[...]
    m = jnp.max(z, axis=1, keepdims=True)
    e = jnp.exp(z - m)
    lse = jnp.log(jnp.sum(e, axis=1, keepdims=True)) + m
    out_ref[...] = z - lse


_f32 = jnp.float32
_MG = 8
_MB = NP // _MG


def _row_spec(w):
    return pl.BlockSpec((_MB, w), lambda i: (i, 0))


def _full_spec(r, w):
    return pl.BlockSpec((r, w), lambda i: (0, 0))


_tc1 = pl.pallas_call(
    _tc1_body,
    grid=(_MG,),
    in_specs=[_row_spec(HW), _row_spec(HW), _row_spec(NFEAT)],
    out_specs=(_row_spec(NFEAT), _row_spec(1)),
    out_shape=(
        jax.ShapeDtypeStruct((NP, NFEAT), _f32),
        jax.ShapeDtypeStruct((NP, 1), _f32),
    ),
)

_tc2 = pl.pallas_call(
    _tc2_body,
    grid=(_MG,),
    in_specs=[_row_spec(NFEAT), _row_spec(NFEAT), _row_spec(NFEAT),
              _row_spec(1), _full_spec(1, NHID), _full_spec(NFEAT, NHID),
              _full_spec(NHID, NCLASS)],
    out_specs=_row_spec(128),
    out_shape=jax.ShapeDtypeStruct((NP, 128), _f32),
)

_tc3 = pl.pallas_call(
    _tc3_body,
    grid=(_MG,),
    in_specs=[_row_spec(128), _row_spec(128), _row_spec(128), _row_spec(1),
              _full_spec(1, NCLASS)],
    out_specs=_row_spec(NCLASS),
    out_shape=jax.ShapeDtypeStruct((NP, NCLASS), _f32),
)


@jax.jit
def kernel(x, adj, W1, b1, W2, b2):
    src = adj[0]
    dst = adj[1]
    pad = N + jnp.arange(EP - E, dtype=jnp.int32) % (NP - N)
    dst2 = jnp.concatenate([dst, pad]).reshape(NBLK, LANES)
    src2b = jnp.concatenate([src, pad]).reshape(NBLK2, BL)
    dst2b = dst2.reshape(NBLK2, BL)

    xp = jnp.concatenate([x, jnp.zeros((NP - N, NFEAT), _f32)], axis=0)

    hist = _sc_degree(dst2)
    gx, dinv = _tc1(hist[:NP], hist[NP:], xp)
    a1 = _agg(gx, src2b, dst2b)
    g2 = _tc2(a1[:NP], a1[NP:], gx, dinv, b1.reshape(1, NHID), W1, W2)
    a2 = _agg(g2, src2b, dst2b)
    out = _tc3(a2[:NP], a2[NP:], g2, dinv, b2.reshape(1, NCLASS))
    return out[:N]
